# baseline (speedup 1.0000x reference)
import functools

import jax
import jax.numpy as jnp
from jax import lax
from jax.experimental import pallas as pl
from jax.experimental.pallas import tpu as pltpu
from jax.experimental.pallas import tpu_sc as plsc

N = 10000
K = 9
C = 128
B = 2
KC = K * C

NC = 2
NS = 16
NW = NC * NS
NPW = 320
NPAD = NW * NPW
NCH = 5
CHW = 64
NPWP = NCH * CHW

BN_CNT = float(B * N)
EPS = 1e-5

MBN = 1000
NB = N // MBN


def _mm_body(x_ref, w_ref, o_ref):
    o_ref[0, 0] = jnp.dot(x_ref[0].astype(jnp.bfloat16), w_ref[0],
                          preferred_element_type=jnp.float32)


def _tc_matmul(x, wk):
    return pl.pallas_call(
        _mm_body,
        grid=(B, NB, K),
        in_specs=[
            pl.BlockSpec((1, MBN, C), lambda b, i, k: (b, i, 0)),
            pl.BlockSpec((1, C, C), lambda b, i, k: (k, 0, 0)),
        ],
        out_specs=pl.BlockSpec((1, 1, MBN, C), lambda b, i, k: (k, b, i, 0)),
        out_shape=jax.ShapeDtypeStruct((K, B, N, C), jnp.float32),
    )(x, wk)


def _stats_body(h_ref, s_ref):
    b = pl.program_id(0)
    i = pl.program_id(1)

    @pl.when(jnp.logical_and(b == 0, i == 0))
    def _():
        s_ref[...] = jnp.zeros_like(s_ref)

    blk = h_ref[0]
    s = jnp.sum(blk, axis=0, keepdims=True)
    sq = jnp.sum(blk * blk, axis=0, keepdims=True)
    s_ref[0:1, :] = s_ref[0:1, :] + s
    s_ref[1:2, :] = s_ref[1:2, :] + sq


def _tc_stats(h):
    return pl.pallas_call(
        _stats_body,
        grid=(B, NB),
        in_specs=[pl.BlockSpec((1, MBN, C), lambda b, i: (b, i, 0))],
        out_specs=pl.BlockSpec((8, C), lambda b, i: (0, 0)),
        out_shape=jax.ShapeDtypeStruct((8, C), jnp.float32),
    )(h)


def _norm_mm_body(h_ref, s_ref, g_ref, be_ref, w_ref, o_ref):
    mean = s_ref[0:1, :] / BN_CNT
    var = s_ref[1:2, :] / BN_CNT - mean * mean
    scale = g_ref[...] * lax.rsqrt(var + EPS)
    shift = be_ref[...] - mean * scale
    a = jnp.maximum(h_ref[0] * scale + shift, 0.0)
    o_ref[0, 0] = jnp.dot(a.astype(jnp.bfloat16), w_ref[0],
                          preferred_element_type=jnp.float32)


def _tc_norm_matmul(h, s, gamma, beta, wk):
    return pl.pallas_call(
        _norm_mm_body,
        grid=(B, NB, K),
        in_specs=[
            pl.BlockSpec((1, MBN, C), lambda b, i, k: (b, i, 0)),
            pl.BlockSpec((8, C), lambda b, i, k: (0, 0)),
            pl.BlockSpec((1, C), lambda b, i, k: (0, 0)),
            pl.BlockSpec((1, C), lambda b, i, k: (0, 0)),
            pl.BlockSpec((1, C, C), lambda b, i, k: (k, 0, 0)),
        ],
        out_specs=pl.BlockSpec((1, 1, MBN, C), lambda b, i, k: (k, b, i, 0)),
        out_shape=jax.ShapeDtypeStruct((K, B, N, C), jnp.float32),
    )(h, s, gamma, beta, wk)


def _final_body(h_ref, x_ref, s_ref, g_ref, be_ref, o_ref):
    mean = s_ref[0:1, :] / BN_CNT
    var = s_ref[1:2, :] / BN_CNT - mean * mean
    scale = g_ref[...] * lax.rsqrt(var + EPS)
    shift = be_ref[...] - mean * scale
    o_ref[0] = jnp.maximum(h_ref[0] * scale + shift + x_ref[0], 0.0)


def _tc_final(h, x, s, gamma, beta):
    return pl.pallas_call(
        _final_body,
        grid=(B, NB),
        in_specs=[
            pl.BlockSpec((1, MBN, C), lambda b, i: (b, i, 0)),
            pl.BlockSpec((1, MBN, C), lambda b, i: (b, i, 0)),
            pl.BlockSpec((8, C), lambda b, i: (0, 0)),
            pl.BlockSpec((1, C), lambda b, i: (0, 0)),
            pl.BlockSpec((1, C), lambda b, i: (0, 0)),
        ],
        out_specs=pl.BlockSpec((1, MBN, C), lambda b, i: (b, i, 0)),
        out_shape=jax.ShapeDtypeStruct((B, N, C), jnp.float32),
    )(h, x, s, gamma, beta)


def _sc_body(yflat, gidx, h_out, idx_v, acc, sem0, sema, semw):
    cid = lax.axis_index("c")
    sid = lax.axis_index("s")
    wid = sid * NC + cid
    base = wid * NPW

    pltpu.sync_copy(gidx.at[wid], idx_v)

    cp0 = [[pltpu.async_copy(
                yflat.at[idx_v.at[b, 0, ch]],
                acc.at[pl.ds(b * NPWP + ch * CHW, CHW)], sem0.at[b, ch])
            for ch in range(NCH)] for b in range(B)]

    wcps = []
    for b in range(B):
        addcps = []
        for ch in range(NCH):
            cp0[b][ch].wait()
            addcps += [pltpu.async_copy(
                           yflat.at[idx_v.at[b, k, ch]],
                           acc.at[pl.ds(b * NPWP + ch * CHW, CHW)], sema,
                           add=True)
                       for k in range(1, K)]
        for cp in addcps:
            cp.wait()
        wcps.append(pltpu.async_copy(acc.at[pl.ds(b * NPWP, NPW)],
                                     h_out.at[b, pl.ds(base, NPW)],
                                     semw.at[b]))
    for cp in wcps:
        cp.wait()


def _sc_gather_sum(yflat, gidx):
    mesh = plsc.VectorSubcoreMesh(core_axis_name="c", subcore_axis_name="s",
                                  num_cores=NC, num_subcores=NS)
    f = pl.kernel(
        _sc_body,
        out_type=jax.ShapeDtypeStruct((B, NPAD, C), jnp.float32),
        mesh=mesh,
        scratch_types=[
            pltpu.VMEM((B, K, NCH, CHW), jnp.int32),
            pltpu.VMEM((B * NPWP, C), jnp.float32),
            pltpu.SemaphoreType.DMA((B, NCH)),
            pltpu.SemaphoreType.DMA,
            pltpu.SemaphoreType.DMA((B,)),
        ],
    )
    return f(yflat, gidx)


def _prep_w(w):
    return w.reshape(K, C, C).astype(jnp.bfloat16)


def _prep_idx(idx):
    idxp = jnp.concatenate(
        [idx, jnp.zeros((NPAD - N, K), jnp.int32)], axis=0)
    kk = jnp.arange(K, dtype=jnp.int32)[None, None, :] * (B * N)
    bb = jnp.arange(B, dtype=jnp.int32)[None, :, None] * N
    g = idxp[:, None, :] + kk + bb
    g = g.reshape(NW, NPW, B, K).transpose(0, 2, 3, 1)
    return g.reshape(NW, B, K, NCH, CHW)


def kernel(x, nbr_idx1, nbr_idx2, W1, b1, gamma1, beta1,
           W2, b2, gamma2, beta2):
    del b1, b2

    w1p = _prep_w(W1)
    w2p = _prep_w(W2)
    g1 = _prep_idx(nbr_idx1)
    g2 = _prep_idx(nbr_idx2)
    gam1 = gamma1.reshape(1, C)
    bet1 = beta1.reshape(1, C)
    gam2 = gamma2.reshape(1, C)
    bet2 = beta2.reshape(1, C)

    y1 = _tc_matmul(x, w1p)
    h1 = _sc_gather_sum(y1.reshape(K * B * N, C), g1)
    s1 = _tc_stats(h1)
    y2 = _tc_norm_matmul(h1, s1, gam1, bet1, w2p)
    h2 = _sc_gather_sum(y2.reshape(K * B * N, C), g2)
    s2 = _tc_stats(h2)
    return _tc_final(h2, x, s2, gam2, bet2)


# speedup vs baseline: 1.0537x; 1.0537x over previous
import functools

import jax
import jax.numpy as jnp
from jax import lax
from jax.experimental import pallas as pl
from jax.experimental.pallas import tpu as pltpu
from jax.experimental.pallas import tpu_sc as plsc

N = 10000
K = 9
C = 128
B = 2
KC = K * C

NC = 2
NS = 16
CHW = 64
NCH0, NCH1 = 6, 4
NPW0, NPW1 = NCH0 * CHW, NCH1 * CHW
NPAD = NS * (NPW0 + NPW1)

BN_CNT = float(B * N)
EPS = 1e-5

MBN = 1000
NB = N // MBN


def _mm_body(x_ref, w_ref, o_ref):
    o_ref[0, 0] = jnp.dot(x_ref[0], w_ref[0],
                          preferred_element_type=jnp.float32)


def _tc_matmul(x, wk):
    return pl.pallas_call(
        _mm_body,
        grid=(B, NB, K),
        in_specs=[
            pl.BlockSpec((1, MBN, C), lambda b, i, k: (b, i, 0)),
            pl.BlockSpec((1, C, C), lambda b, i, k: (k, 0, 0)),
        ],
        out_specs=pl.BlockSpec((1, 1, MBN, C), lambda b, i, k: (k, b, i, 0)),
        out_shape=jax.ShapeDtypeStruct((K, B, N, C), jnp.float32),
    )(x, wk)


def _stats_body(h_ref, s_ref):
    b = pl.program_id(0)
    i = pl.program_id(1)

    @pl.when(jnp.logical_and(b == 0, i == 0))
    def _():
        s_ref[...] = jnp.zeros_like(s_ref)

    blk = h_ref[0]
    s = jnp.sum(blk, axis=0, keepdims=True)
    sq = jnp.sum(blk * blk, axis=0, keepdims=True)
    s_ref[0:1, :] = s_ref[0:1, :] + s
    s_ref[1:2, :] = s_ref[1:2, :] + sq


def _tc_stats(h):
    return pl.pallas_call(
        _stats_body,
        grid=(B, NB),
        in_specs=[pl.BlockSpec((1, MBN, C), lambda b, i: (b, i, 0))],
        out_specs=pl.BlockSpec((8, C), lambda b, i: (0, 0)),
        out_shape=jax.ShapeDtypeStruct((8, C), jnp.float32),
    )(h)


def _norm_mm_body(h_ref, s_ref, g_ref, be_ref, w_ref, o_ref):
    mean = s_ref[0:1, :] / BN_CNT
    var = s_ref[1:2, :] / BN_CNT - mean * mean
    scale = g_ref[...] * lax.rsqrt(var + EPS)
    shift = be_ref[...] - mean * scale
    a = jnp.maximum(h_ref[0] * scale + shift, 0.0)
    o_ref[0, 0] = jnp.dot(a, w_ref[0], preferred_element_type=jnp.float32)


def _tc_norm_matmul(h, s, gamma, beta, wk):
    return pl.pallas_call(
        _norm_mm_body,
        grid=(B, NB, K),
        in_specs=[
            pl.BlockSpec((1, MBN, C), lambda b, i, k: (b, i, 0)),
            pl.BlockSpec((8, C), lambda b, i, k: (0, 0)),
            pl.BlockSpec((1, C), lambda b, i, k: (0, 0)),
            pl.BlockSpec((1, C), lambda b, i, k: (0, 0)),
            pl.BlockSpec((1, C, C), lambda b, i, k: (k, 0, 0)),
        ],
        out_specs=pl.BlockSpec((1, 1, MBN, C), lambda b, i, k: (k, b, i, 0)),
        out_shape=jax.ShapeDtypeStruct((K, B, N, C), jnp.float32),
    )(h, s, gamma, beta, wk)


def _final_body(h_ref, x_ref, s_ref, g_ref, be_ref, o_ref):
    mean = s_ref[0:1, :] / BN_CNT
    var = s_ref[1:2, :] / BN_CNT - mean * mean
    scale = g_ref[...] * lax.rsqrt(var + EPS)
    shift = be_ref[...] - mean * scale
    o_ref[0] = jnp.maximum(h_ref[0] * scale + shift + x_ref[0], 0.0)


def _tc_final(h, x, s, gamma, beta):
    return pl.pallas_call(
        _final_body,
        grid=(B, NB),
        in_specs=[
            pl.BlockSpec((1, MBN, C), lambda b, i: (b, i, 0)),
            pl.BlockSpec((1, MBN, C), lambda b, i: (b, i, 0)),
            pl.BlockSpec((8, C), lambda b, i: (0, 0)),
            pl.BlockSpec((1, C), lambda b, i: (0, 0)),
            pl.BlockSpec((1, C), lambda b, i: (0, 0)),
        ],
        out_specs=pl.BlockSpec((1, MBN, C), lambda b, i: (b, i, 0)),
        out_shape=jax.ShapeDtypeStruct((B, N, C), jnp.float32),
    )(h, x, s, gamma, beta)


def _sc_body(yflat, gidx, h_out, idx_v, acc, sem0, sema, semw):
    cid = lax.axis_index("c")
    sid = lax.axis_index("s")

    pltpu.sync_copy(gidx.at[cid, sid], idx_v)

    def run(nch, npw, base):
        cp0 = [[pltpu.async_copy(
                    yflat.at[idx_v.at[b, 0, ch]],
                    acc.at[pl.ds(b * NPW0 + ch * CHW, CHW)], sem0.at[b, ch])
                for ch in range(nch)] for b in range(B)]

        wcps = []
        for b in range(B):
            addcps = []
            for ch in range(nch):
                cp0[b][ch].wait()
                addcps += [pltpu.async_copy(
                               yflat.at[idx_v.at[b, k, ch]],
                               acc.at[pl.ds(b * NPW0 + ch * CHW, CHW)], sema,
                               add=True)
                           for k in range(1, K)]
            for cp in addcps:
                cp.wait()
            wcps.append(pltpu.async_copy(acc.at[pl.ds(b * NPW0, npw)],
                                         h_out.at[b, pl.ds(base, npw)],
                                         semw.at[b]))
        for cp in wcps:
            cp.wait()

    @pl.when(cid == 0)
    def _():
        run(NCH0, NPW0, sid * NPW0)

    @pl.when(cid == 1)
    def _():
        run(NCH1, NPW1, NS * NPW0 + sid * NPW1)


def _sc_gather_sum(yflat, gidx):
    mesh = plsc.VectorSubcoreMesh(core_axis_name="c", subcore_axis_name="s",
                                  num_cores=NC, num_subcores=NS)
    f = pl.kernel(
        _sc_body,
        out_type=jax.ShapeDtypeStruct((B, NPAD, C), jnp.float32),
        mesh=mesh,
        scratch_types=[
            pltpu.VMEM((B, K, NCH0, CHW), jnp.int32),
            pltpu.VMEM((B * NPW0, C), jnp.float32),
            pltpu.SemaphoreType.DMA((B, NCH0)),
            pltpu.SemaphoreType.DMA,
            pltpu.SemaphoreType.DMA((B,)),
        ],
    )
    return f(yflat, gidx)


def _prep_w(w):
    return w.reshape(K, C, C)


def _prep_idx(idx):
    idxp = jnp.concatenate(
        [idx, jnp.zeros((NPAD - N, K), jnp.int32)], axis=0)
    kk = jnp.arange(K, dtype=jnp.int32)[None, None, :] * (B * N)
    bb = jnp.arange(B, dtype=jnp.int32)[None, :, None] * N
    g = idxp[:, None, :] + kk + bb
    g0 = g[:NS * NPW0].reshape(NS, NPW0, B, K).transpose(0, 2, 3, 1)
    g1 = g[NS * NPW0:].reshape(NS, NPW1, B, K).transpose(0, 2, 3, 1)
    g1 = jnp.concatenate(
        [g1, jnp.zeros((NS, B, K, NPW0 - NPW1), jnp.int32)], axis=-1)
    gs = jnp.stack([g0, g1])
    return gs.reshape(NC, NS, B, K, NCH0, CHW)


def kernel(x, nbr_idx1, nbr_idx2, W1, b1, gamma1, beta1,
           W2, b2, gamma2, beta2):
    del b1, b2

    w1p = _prep_w(W1)
    w2p = _prep_w(W2)
    g1 = _prep_idx(nbr_idx1)
    g2 = _prep_idx(nbr_idx2)
    gam1 = gamma1.reshape(1, C)
    bet1 = beta1.reshape(1, C)
    gam2 = gamma2.reshape(1, C)
    bet2 = beta2.reshape(1, C)

    y1 = _tc_matmul(x, w1p)
    h1 = _sc_gather_sum(y1.reshape(K * B * N, C), g1)
    s1 = _tc_stats(h1)
    y2 = _tc_norm_matmul(h1, s1, gam1, bet1, w2p)
    h2 = _sc_gather_sum(y2.reshape(K * B * N, C), g2)
    s2 = _tc_stats(h2)
    return _tc_final(h2, x, s2, gam2, bet2)


# speedup vs baseline: 1.3850x; 1.3144x over previous
import functools

import jax
import jax.numpy as jnp
from jax import lax
from jax.experimental import pallas as pl
from jax.experimental.pallas import tpu as pltpu
from jax.experimental.pallas import tpu_sc as plsc

N = 10000
K = 9
C = 128
B = 2
KC = K * C

NC = 2
NS = 16
CHW = 64
NCH0, NCH1 = 6, 4
NPW0, NPW1 = NCH0 * CHW, NCH1 * CHW
NPAD = NS * (NPW0 + NPW1)

BN_CNT = float(B * N)
EPS = 1e-5

MBN = 2000
NB = N // MBN


def _mm_body(x_ref, w_ref, o_ref):
    o_ref[0, 0] = jnp.dot(x_ref[0], w_ref[0],
                          preferred_element_type=jnp.float32)


def _tc_matmul(x, wk):
    return pl.pallas_call(
        _mm_body,
        grid=(B, NB, K),
        in_specs=[
            pl.BlockSpec((1, MBN, C), lambda b, i, k: (b, i, 0)),
            pl.BlockSpec((1, C, C), lambda b, i, k: (k, 0, 0)),
        ],
        out_specs=pl.BlockSpec((1, 1, MBN, C), lambda b, i, k: (k, b, i, 0)),
        out_shape=jax.ShapeDtypeStruct((K, B, N, C), jnp.float32),
    )(x, wk)


def _stats_body(h_ref, s_ref):
    b = pl.program_id(0)
    i = pl.program_id(1)

    @pl.when(jnp.logical_and(b == 0, i == 0))
    def _():
        s_ref[...] = jnp.zeros_like(s_ref)

    blk = h_ref[0]
    s = jnp.sum(blk, axis=0, keepdims=True)
    sq = jnp.sum(blk * blk, axis=0, keepdims=True)
    s_ref[0:1, :] = s_ref[0:1, :] + s
    s_ref[1:2, :] = s_ref[1:2, :] + sq


def _tc_stats(h):
    return pl.pallas_call(
        _stats_body,
        grid=(B, NB),
        in_specs=[pl.BlockSpec((1, MBN, C), lambda b, i: (b, i, 0))],
        out_specs=pl.BlockSpec((8, C), lambda b, i: (0, 0)),
        out_shape=jax.ShapeDtypeStruct((8, C), jnp.float32),
    )(h)


def _norm_mm_body(h_ref, s_ref, g_ref, be_ref, w_ref, o_ref):
    mean = s_ref[0:1, :] / BN_CNT
    var = s_ref[1:2, :] / BN_CNT - mean * mean
    scale = g_ref[...] * lax.rsqrt(var + EPS)
    shift = be_ref[...] - mean * scale
    a = jnp.maximum(h_ref[0] * scale + shift, 0.0)
    o_ref[0, 0] = jnp.dot(a, w_ref[0], preferred_element_type=jnp.float32)


def _tc_norm_matmul(h, s, gamma, beta, wk):
    return pl.pallas_call(
        _norm_mm_body,
        grid=(B, NB, K),
        in_specs=[
            pl.BlockSpec((1, MBN, C), lambda b, i, k: (b, i, 0)),
            pl.BlockSpec((8, C), lambda b, i, k: (0, 0)),
            pl.BlockSpec((1, C), lambda b, i, k: (0, 0)),
            pl.BlockSpec((1, C), lambda b, i, k: (0, 0)),
            pl.BlockSpec((1, C, C), lambda b, i, k: (k, 0, 0)),
        ],
        out_specs=pl.BlockSpec((1, 1, MBN, C), lambda b, i, k: (k, b, i, 0)),
        out_shape=jax.ShapeDtypeStruct((K, B, N, C), jnp.float32),
    )(h, s, gamma, beta, wk)


def _final_body(h_ref, x_ref, s_ref, g_ref, be_ref, o_ref):
    mean = s_ref[0:1, :] / BN_CNT
    var = s_ref[1:2, :] / BN_CNT - mean * mean
    scale = g_ref[...] * lax.rsqrt(var + EPS)
    shift = be_ref[...] - mean * scale
    o_ref[0] = jnp.maximum(h_ref[0] * scale + shift + x_ref[0], 0.0)


def _tc_final(h, x, s, gamma, beta):
    return pl.pallas_call(
        _final_body,
        grid=(B, NB),
        in_specs=[
            pl.BlockSpec((1, MBN, C), lambda b, i: (b, i, 0)),
            pl.BlockSpec((1, MBN, C), lambda b, i: (b, i, 0)),
            pl.BlockSpec((8, C), lambda b, i: (0, 0)),
            pl.BlockSpec((1, C), lambda b, i: (0, 0)),
            pl.BlockSpec((1, C), lambda b, i: (0, 0)),
        ],
        out_specs=pl.BlockSpec((1, MBN, C), lambda b, i: (b, i, 0)),
        out_shape=jax.ShapeDtypeStruct((B, N, C), jnp.float32),
    )(h, x, s, gamma, beta)


def _sc_body(yflat, gidx, h_out, idx_v, acc, sem0, sema, semw):
    cid = lax.axis_index("c")
    sid = lax.axis_index("s")

    pltpu.sync_copy(gidx.at[cid, sid], idx_v)

    def run(nch, npw, base):
        cp0 = [[pltpu.async_copy(
                    yflat.at[idx_v.at[b, 0, ch]],
                    acc.at[pl.ds(b * NPW0 + ch * CHW, CHW)], sem0.at[b, ch])
                for ch in range(nch)] for b in range(B)]

        wcps = []
        for b in range(B):
            addcps = []
            for ch in range(nch):
                cp0[b][ch].wait()
                addcps += [pltpu.async_copy(
                               yflat.at[idx_v.at[b, k, ch]],
                               acc.at[pl.ds(b * NPW0 + ch * CHW, CHW)], sema,
                               add=True)
                           for k in range(1, K)]
            for cp in addcps:
                cp.wait()
            wcps.append(pltpu.async_copy(acc.at[pl.ds(b * NPW0, npw)],
                                         h_out.at[b, pl.ds(base, npw)],
                                         semw.at[b]))
        for cp in wcps:
            cp.wait()

    @pl.when(cid == 0)
    def _():
        run(NCH0, NPW0, sid * NPW0)

    @pl.when(cid == 1)
    def _():
        run(NCH1, NPW1, NS * NPW0 + sid * NPW1)


def _sc_gather_sum(yflat, gidx):
    mesh = plsc.VectorSubcoreMesh(core_axis_name="c", subcore_axis_name="s",
                                  num_cores=NC, num_subcores=NS)
    f = pl.kernel(
        _sc_body,
        out_type=jax.ShapeDtypeStruct((B, NPAD, C), jnp.float32),
        mesh=mesh,
        scratch_types=[
            pltpu.VMEM((B, K, NCH0, CHW), jnp.int32),
            pltpu.VMEM((B * NPW0, C), jnp.float32),
            pltpu.SemaphoreType.DMA((B, NCH0)),
            pltpu.SemaphoreType.DMA,
            pltpu.SemaphoreType.DMA((B,)),
        ],
    )
    return f(yflat, gidx)


def _prep_w(w):
    return w.reshape(K, C, C)


def _prep_idx(idx):
    idxp = jnp.concatenate(
        [idx, jnp.zeros((NPAD - N, K), jnp.int32)], axis=0)
    kk = jnp.arange(K, dtype=jnp.int32)[None, None, :] * (B * N)
    bb = jnp.arange(B, dtype=jnp.int32)[None, :, None] * N
    g = idxp[:, None, :] + kk + bb
    g0 = g[:NS * NPW0].reshape(NS, NPW0, B, K).transpose(0, 2, 3, 1)
    g1 = g[NS * NPW0:].reshape(NS, NPW1, B, K).transpose(0, 2, 3, 1)
    g1 = jnp.concatenate(
        [g1, jnp.zeros((NS, B, K, NPW0 - NPW1), jnp.int32)], axis=-1)
    gs = jnp.stack([g0, g1])
    return gs.reshape(NC, NS, B, K, NCH0, CHW)


def kernel(x, nbr_idx1, nbr_idx2, W1, b1, gamma1, beta1,
           W2, b2, gamma2, beta2):
    del b1, b2

    w1p = _prep_w(W1)
    w2p = _prep_w(W2)
    g1 = _prep_idx(nbr_idx1)
    g2 = _prep_idx(nbr_idx2)
    gam1 = gamma1.reshape(1, C)
    bet1 = beta1.reshape(1, C)
    gam2 = gamma2.reshape(1, C)
    bet2 = beta2.reshape(1, C)

    y1 = _tc_matmul(x, w1p)
    h1 = _sc_gather_sum(y1.reshape(K * B * N, C), g1)
    s1 = _tc_stats(h1)
    y2 = _tc_norm_matmul(h1, s1, gam1, bet1, w2p)
    h2 = _sc_gather_sum(y2.reshape(K * B * N, C), g2)
    s2 = _tc_stats(h2)
    return _tc_final(h2, x, s2, gam2, bet2)


# speedup vs baseline: 1.6128x; 1.1645x over previous
import functools

import jax
import jax.numpy as jnp
from jax import lax
from jax.experimental import pallas as pl
from jax.experimental.pallas import tpu as pltpu
from jax.experimental.pallas import tpu_sc as plsc

N = 10000
K = 9
C = 128
B = 2
KC = K * C

NC = 2
NS = 16
CHW = 64
NCH0, NCH1 = 6, 4
NPW0, NPW1 = NCH0 * CHW, NCH1 * CHW
NPAD = NS * (NPW0 + NPW1)

BN_CNT = float(B * N)
EPS = 1e-5

MBN = 5000
NB = N // MBN


def _mm_body(x_ref, w_ref, o_ref):
    o_ref[0, 0] = jnp.dot(x_ref[0], w_ref[0],
                          preferred_element_type=jnp.float32)


def _tc_matmul(x, wk):
    return pl.pallas_call(
        _mm_body,
        grid=(B, NB, K),
        in_specs=[
            pl.BlockSpec((1, MBN, C), lambda b, i, k: (b, i, 0)),
            pl.BlockSpec((1, C, C), lambda b, i, k: (k, 0, 0)),
        ],
        out_specs=pl.BlockSpec((1, 1, MBN, C), lambda b, i, k: (k, b, i, 0)),
        out_shape=jax.ShapeDtypeStruct((K, B, N, C), jnp.float32),
    )(x, wk)


def _stats_body(h_ref, s_ref):
    b = pl.program_id(0)
    i = pl.program_id(1)

    @pl.when(jnp.logical_and(b == 0, i == 0))
    def _():
        s_ref[...] = jnp.zeros_like(s_ref)

    blk = h_ref[0]
    s = jnp.sum(blk, axis=0, keepdims=True)
    sq = jnp.sum(blk * blk, axis=0, keepdims=True)
    s_ref[0:1, :] = s_ref[0:1, :] + s
    s_ref[1:2, :] = s_ref[1:2, :] + sq


def _tc_stats(h):
    return pl.pallas_call(
        _stats_body,
        grid=(B, NB),
        in_specs=[pl.BlockSpec((1, MBN, C), lambda b, i: (b, i, 0))],
        out_specs=pl.BlockSpec((8, C), lambda b, i: (0, 0)),
        out_shape=jax.ShapeDtypeStruct((8, C), jnp.float32),
    )(h)


def _norm_mm_body(h_ref, s_ref, g_ref, be_ref, w_ref, o_ref):
    mean = s_ref[0:1, :] / BN_CNT
    var = s_ref[1:2, :] / BN_CNT - mean * mean
    scale = g_ref[...] * lax.rsqrt(var + EPS)
    shift = be_ref[...] - mean * scale
    a = jnp.maximum(h_ref[0] * scale + shift, 0.0)
    o_ref[0, 0] = jnp.dot(a, w_ref[0], preferred_element_type=jnp.float32)


def _tc_norm_matmul(h, s, gamma, beta, wk):
    return pl.pallas_call(
        _norm_mm_body,
        grid=(B, NB, K),
        in_specs=[
            pl.BlockSpec((1, MBN, C), lambda b, i, k: (b, i, 0)),
            pl.BlockSpec((8, C), lambda b, i, k: (0, 0)),
            pl.BlockSpec((1, C), lambda b, i, k: (0, 0)),
            pl.BlockSpec((1, C), lambda b, i, k: (0, 0)),
            pl.BlockSpec((1, C, C), lambda b, i, k: (k, 0, 0)),
        ],
        out_specs=pl.BlockSpec((1, 1, MBN, C), lambda b, i, k: (k, b, i, 0)),
        out_shape=jax.ShapeDtypeStruct((K, B, N, C), jnp.float32),
    )(h, s, gamma, beta, wk)


def _final_body(h_ref, x_ref, s_ref, g_ref, be_ref, o_ref):
    mean = s_ref[0:1, :] / BN_CNT
    var = s_ref[1:2, :] / BN_CNT - mean * mean
    scale = g_ref[...] * lax.rsqrt(var + EPS)
    shift = be_ref[...] - mean * scale
    o_ref[0] = jnp.maximum(h_ref[0] * scale + shift + x_ref[0], 0.0)


def _tc_final(h, x, s, gamma, beta):
    return pl.pallas_call(
        _final_body,
        grid=(B, NB),
        in_specs=[
            pl.BlockSpec((1, MBN, C), lambda b, i: (b, i, 0)),
            pl.BlockSpec((1, MBN, C), lambda b, i: (b, i, 0)),
            pl.BlockSpec((8, C), lambda b, i: (0, 0)),
            pl.BlockSpec((1, C), lambda b, i: (0, 0)),
            pl.BlockSpec((1, C), lambda b, i: (0, 0)),
        ],
        out_specs=pl.BlockSpec((1, MBN, C), lambda b, i: (b, i, 0)),
        out_shape=jax.ShapeDtypeStruct((B, N, C), jnp.float32),
    )(h, x, s, gamma, beta)


def _sc_body(yflat, gidx, h_out, idx_v, acc, sem0, sema, semw):
    cid = lax.axis_index("c")
    sid = lax.axis_index("s")

    pltpu.sync_copy(gidx.at[cid, sid], idx_v)

    def run(nch, npw, base):
        cp0 = [[pltpu.async_copy(
                    yflat.at[idx_v.at[b, 0, ch]],
                    acc.at[pl.ds(b * NPW0 + ch * CHW, CHW)], sem0.at[b, ch])
                for ch in range(nch)] for b in range(B)]

        wcps = []
        for b in range(B):
            addcps = []
            for ch in range(nch):
                cp0[b][ch].wait()
                addcps += [pltpu.async_copy(
                               yflat.at[idx_v.at[b, k, ch]],
                               acc.at[pl.ds(b * NPW0 + ch * CHW, CHW)], sema,
                               add=True)
                           for k in range(1, K)]
            for cp in addcps:
                cp.wait()
            wcps.append(pltpu.async_copy(acc.at[pl.ds(b * NPW0, npw)],
                                         h_out.at[b, pl.ds(base, npw)],
                                         semw.at[b]))
        for cp in wcps:
            cp.wait()

    @pl.when(cid == 0)
    def _():
        run(NCH0, NPW0, sid * NPW0)

    @pl.when(cid == 1)
    def _():
        run(NCH1, NPW1, NS * NPW0 + sid * NPW1)


def _sc_gather_sum(yflat, gidx):
    mesh = plsc.VectorSubcoreMesh(core_axis_name="c", subcore_axis_name="s",
                                  num_cores=NC, num_subcores=NS)
    f = pl.kernel(
        _sc_body,
        out_type=jax.ShapeDtypeStruct((B, NPAD, C), jnp.float32),
        mesh=mesh,
        scratch_types=[
            pltpu.VMEM((B, K, NCH0, CHW), jnp.int32),
            pltpu.VMEM((B * NPW0, C), jnp.float32),
            pltpu.SemaphoreType.DMA((B, NCH0)),
            pltpu.SemaphoreType.DMA,
            pltpu.SemaphoreType.DMA((B,)),
        ],
    )
    return f(yflat, gidx)


def _prep_w(w):
    return w.reshape(K, C, C)


def _prep_idx(idx):
    idxp = jnp.concatenate(
        [idx, jnp.zeros((NPAD - N, K), jnp.int32)], axis=0)
    kk = jnp.arange(K, dtype=jnp.int32)[None, None, :] * (B * N)
    bb = jnp.arange(B, dtype=jnp.int32)[None, :, None] * N
    g = idxp[:, None, :] + kk + bb
    g0 = g[:NS * NPW0].reshape(NS, NPW0, B, K).transpose(0, 2, 3, 1)
    g1 = g[NS * NPW0:].reshape(NS, NPW1, B, K).transpose(0, 2, 3, 1)
    g1 = jnp.concatenate(
        [g1, jnp.zeros((NS, B, K, NPW0 - NPW1), jnp.int32)], axis=-1)
    gs = jnp.stack([g0, g1])
    return gs.reshape(NC, NS, B, K, NCH0, CHW)


def kernel(x, nbr_idx1, nbr_idx2, W1, b1, gamma1, beta1,
           W2, b2, gamma2, beta2):
    del b1, b2

    w1p = _prep_w(W1)
    w2p = _prep_w(W2)
    g1 = _prep_idx(nbr_idx1)
    g2 = _prep_idx(nbr_idx2)
    gam1 = gamma1.reshape(1, C)
    bet1 = beta1.reshape(1, C)
    gam2 = gamma2.reshape(1, C)
    bet2 = beta2.reshape(1, C)

    y1 = _tc_matmul(x, w1p)
    h1 = _sc_gather_sum(y1.reshape(K * B * N, C), g1)
    s1 = _tc_stats(h1)
    y2 = _tc_norm_matmul(h1, s1, gam1, bet1, w2p)
    h2 = _sc_gather_sum(y2.reshape(K * B * N, C), g2)
    s2 = _tc_stats(h2)
    return _tc_final(h2, x, s2, gam2, bet2)


# speedup vs baseline: 1.7823x; 1.1051x over previous
import functools

import jax
import jax.numpy as jnp
from jax import lax
from jax.experimental import pallas as pl
from jax.experimental.pallas import tpu as pltpu
from jax.experimental.pallas import tpu_sc as plsc

N = 10000
K = 9
C = 128
B = 2
KC = K * C

NC = 2
NS = 16
CHW = 64
NCH0, NCH1 = 6, 4
NPW0, NPW1 = NCH0 * CHW, NCH1 * CHW
NPAD = NS * (NPW0 + NPW1)

BN_CNT = float(B * N)
EPS = 1e-5

MBN = 10000
NB = N // MBN


def _mm_body(x_ref, w_ref, o_ref):
    o_ref[0, 0] = jnp.dot(x_ref[0], w_ref[0],
                          preferred_element_type=jnp.float32)


def _tc_matmul(x, wk):
    return pl.pallas_call(
        _mm_body,
        grid=(B, NB, K),
        in_specs=[
            pl.BlockSpec((1, MBN, C), lambda b, i, k: (b, i, 0)),
            pl.BlockSpec((1, C, C), lambda b, i, k: (k, 0, 0)),
        ],
        out_specs=pl.BlockSpec((1, 1, MBN, C), lambda b, i, k: (k, b, i, 0)),
        out_shape=jax.ShapeDtypeStruct((K, B, N, C), jnp.float32),
    )(x, wk)


def _stats_body(h_ref, s_ref):
    b = pl.program_id(0)
    i = pl.program_id(1)

    @pl.when(jnp.logical_and(b == 0, i == 0))
    def _():
        s_ref[...] = jnp.zeros_like(s_ref)

    blk = h_ref[0]
    s = jnp.sum(blk, axis=0, keepdims=True)
    sq = jnp.sum(blk * blk, axis=0, keepdims=True)
    s_ref[0:1, :] = s_ref[0:1, :] + s
    s_ref[1:2, :] = s_ref[1:2, :] + sq


def _tc_stats(h):
    return pl.pallas_call(
        _stats_body,
        grid=(B, NB),
        in_specs=[pl.BlockSpec((1, MBN, C), lambda b, i: (b, i, 0))],
        out_specs=pl.BlockSpec((8, C), lambda b, i: (0, 0)),
        out_shape=jax.ShapeDtypeStruct((8, C), jnp.float32),
    )(h)


def _norm_mm_body(h_ref, s_ref, g_ref, be_ref, w_ref, o_ref):
    mean = s_ref[0:1, :] / BN_CNT
    var = s_ref[1:2, :] / BN_CNT - mean * mean
    scale = g_ref[...] * lax.rsqrt(var + EPS)
    shift = be_ref[...] - mean * scale
    a = jnp.maximum(h_ref[0] * scale + shift, 0.0)
    o_ref[0, 0] = jnp.dot(a, w_ref[0], preferred_element_type=jnp.float32)


def _tc_norm_matmul(h, s, gamma, beta, wk):
    return pl.pallas_call(
        _norm_mm_body,
        grid=(B, NB, K),
        in_specs=[
            pl.BlockSpec((1, MBN, C), lambda b, i, k: (b, i, 0)),
            pl.BlockSpec((8, C), lambda b, i, k: (0, 0)),
            pl.BlockSpec((1, C), lambda b, i, k: (0, 0)),
            pl.BlockSpec((1, C), lambda b, i, k: (0, 0)),
            pl.BlockSpec((1, C, C), lambda b, i, k: (k, 0, 0)),
        ],
        out_specs=pl.BlockSpec((1, 1, MBN, C), lambda b, i, k: (k, b, i, 0)),
        out_shape=jax.ShapeDtypeStruct((K, B, N, C), jnp.float32),
    )(h, s, gamma, beta, wk)


def _final_body(h_ref, x_ref, s_ref, g_ref, be_ref, o_ref):
    mean = s_ref[0:1, :] / BN_CNT
    var = s_ref[1:2, :] / BN_CNT - mean * mean
    scale = g_ref[...] * lax.rsqrt(var + EPS)
    shift = be_ref[...] - mean * scale
    o_ref[0] = jnp.maximum(h_ref[0] * scale + shift + x_ref[0], 0.0)


def _tc_final(h, x, s, gamma, beta):
    return pl.pallas_call(
        _final_body,
        grid=(B, NB),
        in_specs=[
            pl.BlockSpec((1, MBN, C), lambda b, i: (b, i, 0)),
            pl.BlockSpec((1, MBN, C), lambda b, i: (b, i, 0)),
            pl.BlockSpec((8, C), lambda b, i: (0, 0)),
            pl.BlockSpec((1, C), lambda b, i: (0, 0)),
            pl.BlockSpec((1, C), lambda b, i: (0, 0)),
        ],
        out_specs=pl.BlockSpec((1, MBN, C), lambda b, i: (b, i, 0)),
        out_shape=jax.ShapeDtypeStruct((B, N, C), jnp.float32),
    )(h, x, s, gamma, beta)


def _sc_body(yflat, gidx, h_out, idx_v, acc, sem0, sema, semw):
    cid = lax.axis_index("c")
    sid = lax.axis_index("s")

    pltpu.sync_copy(gidx.at[cid, sid], idx_v)

    def run(nch, npw, base):
        cp0 = [[pltpu.async_copy(
                    yflat.at[idx_v.at[b, 0, ch]],
                    acc.at[pl.ds(b * NPW0 + ch * CHW, CHW)], sem0.at[b, ch])
                for ch in range(nch)] for b in range(B)]

        wcps = []
        for b in range(B):
            addcps = []
            for ch in range(nch):
                cp0[b][ch].wait()
                addcps += [pltpu.async_copy(
                               yflat.at[idx_v.at[b, k, ch]],
                               acc.at[pl.ds(b * NPW0 + ch * CHW, CHW)], sema,
                               add=True)
                           for k in range(1, K)]
            for cp in addcps:
                cp.wait()
            wcps.append(pltpu.async_copy(acc.at[pl.ds(b * NPW0, npw)],
                                         h_out.at[b, pl.ds(base, npw)],
                                         semw.at[b]))
        for cp in wcps:
            cp.wait()

    @pl.when(cid == 0)
    def _():
        run(NCH0, NPW0, sid * NPW0)

    @pl.when(cid == 1)
    def _():
        run(NCH1, NPW1, NS * NPW0 + sid * NPW1)


def _sc_gather_sum(yflat, gidx):
    mesh = plsc.VectorSubcoreMesh(core_axis_name="c", subcore_axis_name="s",
                                  num_cores=NC, num_subcores=NS)
    f = pl.kernel(
        _sc_body,
        out_type=jax.ShapeDtypeStruct((B, NPAD, C), jnp.float32),
        mesh=mesh,
        scratch_types=[
            pltpu.VMEM((B, K, NCH0, CHW), jnp.int32),
            pltpu.VMEM((B * NPW0, C), jnp.float32),
            pltpu.SemaphoreType.DMA((B, NCH0)),
            pltpu.SemaphoreType.DMA,
            pltpu.SemaphoreType.DMA((B,)),
        ],
    )
    return f(yflat, gidx)


def _prep_w(w):
    return w.reshape(K, C, C)


def _prep_idx(idx):
    idxp = jnp.concatenate(
        [idx, jnp.zeros((NPAD - N, K), jnp.int32)], axis=0)
    kk = jnp.arange(K, dtype=jnp.int32)[None, None, :] * (B * N)
    bb = jnp.arange(B, dtype=jnp.int32)[None, :, None] * N
    g = idxp[:, None, :] + kk + bb
    g0 = g[:NS * NPW0].reshape(NS, NPW0, B, K).transpose(0, 2, 3, 1)
    g1 = g[NS * NPW0:].reshape(NS, NPW1, B, K).transpose(0, 2, 3, 1)
    g1 = jnp.concatenate(
        [g1, jnp.zeros((NS, B, K, NPW0 - NPW1), jnp.int32)], axis=-1)
    gs = jnp.stack([g0, g1])
    return gs.reshape(NC, NS, B, K, NCH0, CHW)


def kernel(x, nbr_idx1, nbr_idx2, W1, b1, gamma1, beta1,
           W2, b2, gamma2, beta2):
    del b1, b2

    w1p = _prep_w(W1)
    w2p = _prep_w(W2)
    g1 = _prep_idx(nbr_idx1)
    g2 = _prep_idx(nbr_idx2)
    gam1 = gamma1.reshape(1, C)
    bet1 = beta1.reshape(1, C)
    gam2 = gamma2.reshape(1, C)
    bet2 = beta2.reshape(1, C)

    y1 = _tc_matmul(x, w1p)
    h1 = _sc_gather_sum(y1.reshape(K * B * N, C), g1)
    s1 = _tc_stats(h1)
    y2 = _tc_norm_matmul(h1, s1, gam1, bet1, w2p)
    h2 = _sc_gather_sum(y2.reshape(K * B * N, C), g2)
    s2 = _tc_stats(h2)
    return _tc_final(h2, x, s2, gam2, bet2)
